# Initial kernel scaffold; baseline (speedup 1.0000x reference)
import jax
import jax.numpy as jnp
from jax.experimental import pallas as pl


def kernel(x, edge_index, W_l1, b1, W_r1, W_l2, b2, W_r2):
    raise NotImplementedError("write your pallas kernel here")


# speedup vs baseline: 10.3547x; 10.3547x over previous
import functools

import jax
import jax.numpy as jnp
from jax import lax
from jax.experimental import pallas as pl
from jax.experimental.pallas import tpu as pltpu
from jax.experimental.pallas import tpu_sc as plsc

N = 10000
D_IN = 128
D_HID = 16
D_OUT = 7

NC = 2
NS = 16
NW = NC * NS

NP = 10240
EB = 128
E_PAD = 2560 * EB
RW = 2560 // NW
ROWS_PER_TILE = NP // NS


def _sc_aggregate(with_counts):
  out_type = [jax.ShapeDtypeStruct((NC, NP, D_HID), jnp.float32)]
  if with_counts:
    out_type.append(jax.ShapeDtypeStruct((NC, NP), jnp.float32))

  scratch = [
      pltpu.VMEM((RW, EB), jnp.int32),
      pltpu.VMEM((RW, EB), jnp.int32),
      pltpu.VMEM((EB, D_HID), jnp.float32),
      pltpu.VMEM((EB,), jnp.float32),
      pltpu.VMEM_SHARED((NP, D_HID), jnp.float32),
      pltpu.VMEM_SHARED((NP,), jnp.float32),
      pltpu.SemaphoreType.DMA,
  ]

  mesh = plsc.VectorSubcoreMesh(
      core_axis_name="c", subcore_axis_name="s",
      num_cores=NC, num_subcores=NS)

  @functools.partial(pl.kernel, out_type=out_type, mesh=mesh,
                     scratch_types=scratch,
                     compiler_params=pltpu.CompilerParams(
                         use_tc_tiling_on_sc=False))
  def body(src_hbm, dst_hbm, tbl_hbm, z2_hbm, z1_hbm, ones_hbm,
           agg_hbm, *rest):
    if with_counts:
      cnt_hbm = rest[0]
      rest = rest[1:]
    idx_s, idx_d, rows, ones_v, acc, cnta, sem = rest

    cid = lax.axis_index("c")
    sid = lax.axis_index("s")
    wid = cid * NS + sid

    r0 = sid * ROWS_PER_TILE
    pltpu.sync_copy(z2_hbm.at[pl.ds(r0, ROWS_PER_TILE)],
                    acc.at[pl.ds(r0, ROWS_PER_TILE)])
    if with_counts:
      pltpu.sync_copy(z1_hbm.at[pl.ds(r0, ROWS_PER_TILE)],
                      cnta.at[pl.ds(r0, ROWS_PER_TILE)])
      pltpu.sync_copy(ones_hbm, ones_v)
    plsc.subcore_barrier()

    pltpu.sync_copy(src_hbm.at[pl.ds(wid * RW, RW)], idx_s)
    pltpu.sync_copy(dst_hbm.at[pl.ds(wid * RW, RW)], idx_d)

    def step(j, _):
      pltpu.async_copy(tbl_hbm.at[idx_s.at[j]], rows, sem).wait()
      pltpu.sync_copy(rows, acc.at[idx_d.at[j]], add=True)
      if with_counts:
        pltpu.sync_copy(ones_v, cnta.at[idx_d.at[j]], add=True)
      return 0

    lax.fori_loop(0, RW, step, 0)
    plsc.subcore_barrier()

    pltpu.sync_copy(acc.at[pl.ds(r0, ROWS_PER_TILE)],
                    agg_hbm.at[cid, pl.ds(r0, ROWS_PER_TILE)])
    if with_counts:
      pltpu.sync_copy(cnta.at[pl.ds(r0, ROWS_PER_TILE)],
                      cnt_hbm.at[cid, pl.ds(r0, ROWS_PER_TILE)])

  return body


_sc_agg_counts = _sc_aggregate(True)
_sc_agg = _sc_aggregate(False)


def _tc_proj_kernel(x_ref, w_ref, p_ref, r_ref):
  y = jnp.dot(x_ref[...], w_ref[...], preferred_element_type=jnp.float32)
  p_ref[...] = y[:, :D_HID]
  r_ref[...] = y[:, D_HID:]


def _tc_mid_kernel(agg_ref, cnt_ref, r1_ref, b1_ref, w2_ref,
                   p2_ref, r2_ref, iv_ref):
  agg = agg_ref[0] + agg_ref[1]
  cnt = cnt_ref[0] + cnt_ref[1]
  iv = 1.0 / jnp.maximum(cnt, 1.0)
  h = jnp.maximum(agg * iv + b1_ref[...] + r1_ref[...], 0.0)
  y = jnp.dot(h, w2_ref[...], preferred_element_type=jnp.float32)
  p2_ref[...] = y[:, :D_HID]
  r2_ref[...] = y[:, D_HID:]
  iv_ref[...] = iv


def _tc_out_kernel(agg_ref, iv_ref, r2_ref, b2_ref, o_ref):
  z = (agg_ref[0] + agg_ref[1]) * iv_ref[...] + b2_ref[...] + r2_ref[...]
  mask = lax.broadcasted_iota(jnp.int32, z.shape, 1) < D_OUT
  zm = jnp.where(mask, z, -jnp.inf)
  m = jnp.max(zm, axis=1, keepdims=True)
  e = jnp.where(mask, jnp.exp(z - m), 0.0)
  lse = jnp.log(jnp.sum(e, axis=1, keepdims=True))
  o_ref[...] = (z - m - lse)[:, :D_OUT]


_BN = 1000


def kernel(x, edge_index, W_l1, b1, W_r1, W_l2, b2, W_r2):
  f32 = jnp.float32
  src = edge_index[0]
  dst = edge_index[1]

  pad = E_PAD - src.shape[0]
  src_p = jnp.concatenate([src, jnp.zeros((pad,), jnp.int32)])
  dst_p = jnp.concatenate([dst, jnp.full((pad,), NP - 1, jnp.int32)])
  src2d = src_p.reshape(E_PAD // EB, EB)
  dst2d = dst_p.reshape(E_PAD // EB, EB)

  z2 = jnp.zeros((NP, D_HID), f32)
  z1 = jnp.zeros((NP,), f32)
  ones = jnp.ones((EB,), f32)

  w1 = jnp.concatenate([W_l1, W_r1], axis=1)
  grid = N // _BN
  p1, r1 = pl.pallas_call(
      _tc_proj_kernel,
      grid=(grid,),
      in_specs=[
          pl.BlockSpec((_BN, D_IN), lambda i: (i, 0)),
          pl.BlockSpec((D_IN, 2 * D_HID), lambda i: (0, 0)),
      ],
      out_specs=[
          pl.BlockSpec((_BN, D_HID), lambda i: (i, 0)),
          pl.BlockSpec((_BN, D_HID), lambda i: (i, 0)),
      ],
      out_shape=[
          jax.ShapeDtypeStruct((N, D_HID), f32),
          jax.ShapeDtypeStruct((N, D_HID), f32),
      ],
  )(x, w1)

  agg1, cnt1 = _sc_agg_counts(src2d, dst2d, p1, z2, z1, ones)

  wl2p = jnp.zeros((D_HID, D_HID), f32).at[:, :D_OUT].set(W_l2)
  wr2p = jnp.zeros((D_HID, D_HID), f32).at[:, :D_OUT].set(W_r2)
  w2 = jnp.concatenate([wl2p, wr2p], axis=1)
  b1r = b1.reshape(1, D_HID)
  p2, r2, iv = pl.pallas_call(
      _tc_mid_kernel,
      grid=(grid,),
      in_specs=[
          pl.BlockSpec((NC, _BN, D_HID), lambda i: (0, i, 0)),
          pl.BlockSpec((NC, _BN, 1), lambda i: (0, i, 0)),
          pl.BlockSpec((_BN, D_HID), lambda i: (i, 0)),
          pl.BlockSpec((1, D_HID), lambda i: (0, 0)),
          pl.BlockSpec((D_HID, 2 * D_HID), lambda i: (0, 0)),
      ],
      out_specs=[
          pl.BlockSpec((_BN, D_HID), lambda i: (i, 0)),
          pl.BlockSpec((_BN, D_HID), lambda i: (i, 0)),
          pl.BlockSpec((_BN, 1), lambda i: (i, 0)),
      ],
      out_shape=[
          jax.ShapeDtypeStruct((N, D_HID), f32),
          jax.ShapeDtypeStruct((N, D_HID), f32),
          jax.ShapeDtypeStruct((N, 1), f32),
      ],
  )(agg1[:, :N], cnt1[:, :N, None], r1, b1r, w2)

  agg2 = _sc_agg(src2d, dst2d, p2, z2, z1, ones)
  if isinstance(agg2, (list, tuple)):
    agg2 = agg2[0]

  b2r = jnp.zeros((1, D_HID), f32).at[0, :D_OUT].set(b2)
  out = pl.pallas_call(
      _tc_out_kernel,
      grid=(grid,),
      in_specs=[
          pl.BlockSpec((NC, _BN, D_HID), lambda i: (0, i, 0)),
          pl.BlockSpec((_BN, 1), lambda i: (i, 0)),
          pl.BlockSpec((_BN, D_HID), lambda i: (i, 0)),
          pl.BlockSpec((1, D_HID), lambda i: (0, 0)),
      ],
      out_specs=pl.BlockSpec((_BN, D_OUT), lambda i: (i, 0)),
      out_shape=jax.ShapeDtypeStruct((N, D_OUT), f32),
  )(agg2[:, :N], iv, r2, b2r)

  return out


# speedup vs baseline: 11.2458x; 1.0861x over previous
import functools

import jax
import jax.numpy as jnp
from jax import lax
from jax.experimental import pallas as pl
from jax.experimental.pallas import tpu as pltpu
from jax.experimental.pallas import tpu_sc as plsc

N = 10000
D_IN = 128
D_HID = 16
D_OUT = 7

NC = 2
NS = 16
NW = NC * NS

NP = 10240
EB = 128
E_PAD = 2560 * EB
RW = 2560 // NW
ROWS_PER_TILE = NP // NS


def _sc_aggregate(with_counts):
  out_type = [jax.ShapeDtypeStruct((NC, NP, D_HID), jnp.float32)]
  if with_counts:
    out_type.append(jax.ShapeDtypeStruct((NC, NP), jnp.float32))

  scratch = [
      pltpu.VMEM((RW, EB), jnp.int32),
      pltpu.VMEM((RW, EB), jnp.int32),
      pltpu.VMEM((2, EB, D_HID), jnp.float32),
      pltpu.VMEM((EB,), jnp.float32),
      pltpu.VMEM_SHARED((NP, D_HID), jnp.float32),
      pltpu.VMEM_SHARED((NP,), jnp.float32),
      pltpu.SemaphoreType.DMA,
      pltpu.SemaphoreType.DMA,
      pltpu.SemaphoreType.DMA,
  ]

  mesh = plsc.VectorSubcoreMesh(
      core_axis_name="c", subcore_axis_name="s",
      num_cores=NC, num_subcores=NS)

  @functools.partial(pl.kernel, out_type=out_type, mesh=mesh,
                     scratch_types=scratch,
                     compiler_params=pltpu.CompilerParams(
                         use_tc_tiling_on_sc=False))
  def body(src_hbm, dst_hbm, tbl_hbm, z2_hbm, z1_hbm, ones_hbm,
           agg_hbm, *rest):
    if with_counts:
      cnt_hbm = rest[0]
      rest = rest[1:]
    idx_s, idx_d, rows, ones_v, acc, cnta, semg, sems, semc = rest

    cid = lax.axis_index("c")
    sid = lax.axis_index("s")
    wid = cid * NS + sid

    r0 = sid * ROWS_PER_TILE
    pltpu.sync_copy(z2_hbm.at[pl.ds(r0, ROWS_PER_TILE)],
                    acc.at[pl.ds(r0, ROWS_PER_TILE)])
    if with_counts:
      pltpu.sync_copy(z1_hbm.at[pl.ds(r0, ROWS_PER_TILE)],
                      cnta.at[pl.ds(r0, ROWS_PER_TILE)])
      pltpu.sync_copy(ones_hbm, ones_v)
    plsc.subcore_barrier()

    pltpu.sync_copy(src_hbm.at[pl.ds(wid * RW, RW)], idx_s)
    pltpu.sync_copy(dst_hbm.at[pl.ds(wid * RW, RW)], idx_d)

    def fire_gather(j, slot):
      pltpu.async_copy(tbl_hbm.at[idx_s.at[j]], rows.at[slot], semg)

    def wait_gather():
      pltpu.make_async_copy(tbl_hbm.at[idx_s.at[0]], rows.at[0], semg).wait()

    def fire_scatter(j, slot):
      pltpu.async_copy(rows.at[slot], acc.at[idx_d.at[j]], sems, add=True)
      if with_counts:
        pltpu.async_copy(ones_v, cnta.at[idx_d.at[j]], semc, add=True)

    def wait_scatter():
      pltpu.make_async_copy(rows.at[0], acc.at[idx_d.at[0]], sems).wait()
      if with_counts:
        pltpu.make_async_copy(ones_v, cnta.at[idx_d.at[0]], semc).wait()

    fire_gather(0, 0)

    def step(j, _):
      slot = lax.rem(j, 2)
      wait_gather()

      @pl.when(j >= 1)
      def _():
        wait_scatter()

      @pl.when(j + 1 < RW)
      def _():
        fire_gather(j + 1, 1 - slot)

      fire_scatter(j, slot)
      return 0

    lax.fori_loop(0, RW, step, 0)
    wait_scatter()
    plsc.subcore_barrier()

    pltpu.sync_copy(acc.at[pl.ds(r0, ROWS_PER_TILE)],
                    agg_hbm.at[cid, pl.ds(r0, ROWS_PER_TILE)])
    if with_counts:
      pltpu.sync_copy(cnta.at[pl.ds(r0, ROWS_PER_TILE)],
                      cnt_hbm.at[cid, pl.ds(r0, ROWS_PER_TILE)])

  return body


_sc_agg_counts = _sc_aggregate(True)
_sc_agg = _sc_aggregate(False)


def _tc_proj_kernel(x_ref, w_ref, p_ref, r_ref):
  y = jnp.dot(x_ref[...], w_ref[...], preferred_element_type=jnp.float32)
  p_ref[...] = y[:, :D_HID]
  r_ref[...] = y[:, D_HID:]


def _tc_mid_kernel(agg_ref, cnt_ref, r1_ref, b1_ref, w2_ref,
                   p2_ref, r2_ref, iv_ref):
  agg = agg_ref[0] + agg_ref[1]
  cnt = cnt_ref[0] + cnt_ref[1]
  iv = 1.0 / jnp.maximum(cnt, 1.0)
  h = jnp.maximum(agg * iv + b1_ref[...] + r1_ref[...], 0.0)
  y = jnp.dot(h, w2_ref[...], preferred_element_type=jnp.float32)
  p2_ref[...] = y[:, :D_HID]
  r2_ref[...] = y[:, D_HID:]
  iv_ref[...] = iv


def _tc_out_kernel(agg_ref, iv_ref, r2_ref, b2_ref, o_ref):
  z = (agg_ref[0] + agg_ref[1]) * iv_ref[...] + b2_ref[...] + r2_ref[...]
  mask = lax.broadcasted_iota(jnp.int32, z.shape, 1) < D_OUT
  zm = jnp.where(mask, z, -jnp.inf)
  m = jnp.max(zm, axis=1, keepdims=True)
  e = jnp.where(mask, jnp.exp(z - m), 0.0)
  lse = jnp.log(jnp.sum(e, axis=1, keepdims=True))
  o_ref[...] = (z - m - lse)[:, :D_OUT]


_BN = 1000


def kernel(x, edge_index, W_l1, b1, W_r1, W_l2, b2, W_r2):
  f32 = jnp.float32
  src = edge_index[0]
  dst = edge_index[1]

  pad = E_PAD - src.shape[0]
  src_p = jnp.concatenate([src, jnp.zeros((pad,), jnp.int32)])
  dst_p = jnp.concatenate([dst, jnp.full((pad,), NP - 1, jnp.int32)])
  src2d = src_p.reshape(E_PAD // EB, EB)
  dst2d = dst_p.reshape(E_PAD // EB, EB)

  z2 = jnp.zeros((NP, D_HID), f32)
  z1 = jnp.zeros((NP,), f32)
  ones = jnp.ones((EB,), f32)

  w1 = jnp.concatenate([W_l1, W_r1], axis=1)
  grid = N // _BN
  p1, r1 = pl.pallas_call(
      _tc_proj_kernel,
      grid=(grid,),
      in_specs=[
          pl.BlockSpec((_BN, D_IN), lambda i: (i, 0)),
          pl.BlockSpec((D_IN, 2 * D_HID), lambda i: (0, 0)),
      ],
      out_specs=[
          pl.BlockSpec((_BN, D_HID), lambda i: (i, 0)),
          pl.BlockSpec((_BN, D_HID), lambda i: (i, 0)),
      ],
      out_shape=[
          jax.ShapeDtypeStruct((N, D_HID), f32),
          jax.ShapeDtypeStruct((N, D_HID), f32),
      ],
  )(x, w1)

  agg1, cnt1 = _sc_agg_counts(src2d, dst2d, p1, z2, z1, ones)

  wl2p = jnp.zeros((D_HID, D_HID), f32).at[:, :D_OUT].set(W_l2)
  wr2p = jnp.zeros((D_HID, D_HID), f32).at[:, :D_OUT].set(W_r2)
  w2 = jnp.concatenate([wl2p, wr2p], axis=1)
  b1r = b1.reshape(1, D_HID)
  p2, r2, iv = pl.pallas_call(
      _tc_mid_kernel,
      grid=(grid,),
      in_specs=[
          pl.BlockSpec((NC, _BN, D_HID), lambda i: (0, i, 0)),
          pl.BlockSpec((NC, _BN, 1), lambda i: (0, i, 0)),
          pl.BlockSpec((_BN, D_HID), lambda i: (i, 0)),
          pl.BlockSpec((1, D_HID), lambda i: (0, 0)),
          pl.BlockSpec((D_HID, 2 * D_HID), lambda i: (0, 0)),
      ],
      out_specs=[
          pl.BlockSpec((_BN, D_HID), lambda i: (i, 0)),
          pl.BlockSpec((_BN, D_HID), lambda i: (i, 0)),
          pl.BlockSpec((_BN, 1), lambda i: (i, 0)),
      ],
      out_shape=[
          jax.ShapeDtypeStruct((N, D_HID), f32),
          jax.ShapeDtypeStruct((N, D_HID), f32),
          jax.ShapeDtypeStruct((N, 1), f32),
      ],
  )(agg1[:, :N], cnt1[:, :N, None], r1, b1r, w2)

  agg2 = _sc_agg(src2d, dst2d, p2, z2, z1, ones)
  if isinstance(agg2, (list, tuple)):
    agg2 = agg2[0]

  b2r = jnp.zeros((1, D_HID), f32).at[0, :D_OUT].set(b2)
  out = pl.pallas_call(
      _tc_out_kernel,
      grid=(grid,),
      in_specs=[
          pl.BlockSpec((NC, _BN, D_HID), lambda i: (0, i, 0)),
          pl.BlockSpec((_BN, 1), lambda i: (i, 0)),
          pl.BlockSpec((_BN, D_HID), lambda i: (i, 0)),
          pl.BlockSpec((1, D_HID), lambda i: (0, 0)),
      ],
      out_specs=pl.BlockSpec((_BN, D_OUT), lambda i: (i, 0)),
      out_shape=jax.ShapeDtypeStruct((N, D_OUT), f32),
  )(agg2[:, :N], iv, r2, b2r)

  return out


# speedup vs baseline: 19.9476x; 1.7738x over previous
import functools

import jax
import jax.numpy as jnp
from jax import lax
from jax.experimental import pallas as pl
from jax.experimental.pallas import tpu as pltpu
from jax.experimental.pallas import tpu_sc as plsc

N = 10000
D_IN = 128
D_HID = 16
D_OUT = 7

NC = 2
NS = 16
NW = NC * NS

NP = 10240
EB = 128
E_PAD = 2560 * EB
RW = 2560 // NW
ROWS_PER_TILE = NP // NS


def _sc_aggregate(with_counts):
  out_type = [jax.ShapeDtypeStruct((NC, NP, D_HID), jnp.float32)]
  if with_counts:
    out_type.append(jax.ShapeDtypeStruct((NC, NP), jnp.float32))

  scratch = [
      pltpu.VMEM((RW, EB), jnp.int32),
      pltpu.VMEM((RW, EB), jnp.int32),
      pltpu.VMEM((2, EB, D_HID), jnp.float32),
      pltpu.VMEM((EB,), jnp.float32),
      pltpu.VMEM_SHARED((NP, D_HID), jnp.float32),
      pltpu.VMEM_SHARED((NP,), jnp.float32),
      pltpu.VMEM_SHARED((N, D_HID), jnp.float32),
      pltpu.SemaphoreType.DMA,
      pltpu.SemaphoreType.DMA,
      pltpu.SemaphoreType.DMA,
  ]

  mesh = plsc.VectorSubcoreMesh(
      core_axis_name="c", subcore_axis_name="s",
      num_cores=NC, num_subcores=NS)

  @functools.partial(pl.kernel, out_type=out_type, mesh=mesh,
                     scratch_types=scratch,
                     compiler_params=pltpu.CompilerParams(
                         use_tc_tiling_on_sc=False))
  def body(src_hbm, dst_hbm, tbl_hbm, z2_hbm, z1_hbm, ones_hbm,
           agg_hbm, *rest):
    if with_counts:
      cnt_hbm = rest[0]
      rest = rest[1:]
    idx_s, idx_d, rows, ones_v, acc, cnta, tbls, semg, sems, semc = rest

    cid = lax.axis_index("c")
    sid = lax.axis_index("s")
    wid = cid * NS + sid

    r0 = sid * ROWS_PER_TILE
    t0 = sid * (N // NS)
    pltpu.sync_copy(tbl_hbm.at[pl.ds(t0, N // NS)],
                    tbls.at[pl.ds(t0, N // NS)])
    pltpu.sync_copy(z2_hbm.at[pl.ds(r0, ROWS_PER_TILE)],
                    acc.at[pl.ds(r0, ROWS_PER_TILE)])
    if with_counts:
      pltpu.sync_copy(z1_hbm.at[pl.ds(r0, ROWS_PER_TILE)],
                      cnta.at[pl.ds(r0, ROWS_PER_TILE)])
      pltpu.sync_copy(ones_hbm, ones_v)
    plsc.subcore_barrier()

    pltpu.sync_copy(src_hbm.at[pl.ds(wid * RW, RW)], idx_s)
    pltpu.sync_copy(dst_hbm.at[pl.ds(wid * RW, RW)], idx_d)

    def fire_gather(j, slot):
      pltpu.async_copy(tbls.at[idx_s.at[j]], rows.at[slot], semg)

    def wait_gather():
      pltpu.make_async_copy(tbls.at[idx_s.at[0]], rows.at[0], semg).wait()

    def fire_scatter(j, slot):
      pltpu.async_copy(rows.at[slot], acc.at[idx_d.at[j]], sems, add=True)
      if with_counts:
        pltpu.async_copy(ones_v, cnta.at[idx_d.at[j]], semc, add=True)

    def wait_scatter():
      pltpu.make_async_copy(rows.at[0], acc.at[idx_d.at[0]], sems).wait()
      if with_counts:
        pltpu.make_async_copy(ones_v, cnta.at[idx_d.at[0]], semc).wait()

    fire_gather(0, 0)

    def step(j, _):
      slot = lax.rem(j, 2)
      wait_gather()

      @pl.when(j >= 1)
      def _():
        wait_scatter()

      @pl.when(j + 1 < RW)
      def _():
        fire_gather(j + 1, 1 - slot)

      fire_scatter(j, slot)
      return 0

    lax.fori_loop(0, RW, step, 0)
    wait_scatter()
    plsc.subcore_barrier()

    pltpu.sync_copy(acc.at[pl.ds(r0, ROWS_PER_TILE)],
                    agg_hbm.at[cid, pl.ds(r0, ROWS_PER_TILE)])
    if with_counts:
      pltpu.sync_copy(cnta.at[pl.ds(r0, ROWS_PER_TILE)],
                      cnt_hbm.at[cid, pl.ds(r0, ROWS_PER_TILE)])

  return body


_sc_agg_counts = _sc_aggregate(True)
_sc_agg = _sc_aggregate(False)


def _tc_proj_kernel(x_ref, w_ref, p_ref, r_ref):
  y = jnp.dot(x_ref[...], w_ref[...], preferred_element_type=jnp.float32)
  p_ref[...] = y[:, :D_HID]
  r_ref[...] = y[:, D_HID:]


def _tc_mid_kernel(agg_ref, cnt_ref, r1_ref, b1_ref, w2_ref,
                   p2_ref, r2_ref, iv_ref):
  agg = agg_ref[0] + agg_ref[1]
  cnt = cnt_ref[0] + cnt_ref[1]
  iv = 1.0 / jnp.maximum(cnt, 1.0)
  h = jnp.maximum(agg * iv + b1_ref[...] + r1_ref[...], 0.0)
  y = jnp.dot(h, w2_ref[...], preferred_element_type=jnp.float32)
  p2_ref[...] = y[:, :D_HID]
  r2_ref[...] = y[:, D_HID:]
  iv_ref[...] = iv


def _tc_out_kernel(agg_ref, iv_ref, r2_ref, b2_ref, o_ref):
  z = (agg_ref[0] + agg_ref[1]) * iv_ref[...] + b2_ref[...] + r2_ref[...]
  mask = lax.broadcasted_iota(jnp.int32, z.shape, 1) < D_OUT
  zm = jnp.where(mask, z, -jnp.inf)
  m = jnp.max(zm, axis=1, keepdims=True)
  e = jnp.where(mask, jnp.exp(z - m), 0.0)
  lse = jnp.log(jnp.sum(e, axis=1, keepdims=True))
  o_ref[...] = (z - m - lse)[:, :D_OUT]


_BN = 1000


def kernel(x, edge_index, W_l1, b1, W_r1, W_l2, b2, W_r2):
  f32 = jnp.float32
  src = edge_index[0]
  dst = edge_index[1]

  pad = E_PAD - src.shape[0]
  src_p = jnp.concatenate([src, jnp.zeros((pad,), jnp.int32)])
  dst_p = jnp.concatenate([dst, jnp.full((pad,), NP - 1, jnp.int32)])
  src2d = src_p.reshape(E_PAD // EB, EB)
  dst2d = dst_p.reshape(E_PAD // EB, EB)

  z2 = jnp.zeros((NP, D_HID), f32)
  z1 = jnp.zeros((NP,), f32)
  ones = jnp.ones((EB,), f32)

  w1 = jnp.concatenate([W_l1, W_r1], axis=1)
  grid = N // _BN
  p1, r1 = pl.pallas_call(
      _tc_proj_kernel,
      grid=(grid,),
      in_specs=[
          pl.BlockSpec((_BN, D_IN), lambda i: (i, 0)),
          pl.BlockSpec((D_IN, 2 * D_HID), lambda i: (0, 0)),
      ],
      out_specs=[
          pl.BlockSpec((_BN, D_HID), lambda i: (i, 0)),
          pl.BlockSpec((_BN, D_HID), lambda i: (i, 0)),
      ],
      out_shape=[
          jax.ShapeDtypeStruct((N, D_HID), f32),
          jax.ShapeDtypeStruct((N, D_HID), f32),
      ],
  )(x, w1)

  agg1, cnt1 = _sc_agg_counts(src2d, dst2d, p1, z2, z1, ones)

  wl2p = jnp.zeros((D_HID, D_HID), f32).at[:, :D_OUT].set(W_l2)
  wr2p = jnp.zeros((D_HID, D_HID), f32).at[:, :D_OUT].set(W_r2)
  w2 = jnp.concatenate([wl2p, wr2p], axis=1)
  b1r = b1.reshape(1, D_HID)
  p2, r2, iv = pl.pallas_call(
      _tc_mid_kernel,
      grid=(grid,),
      in_specs=[
          pl.BlockSpec((NC, _BN, D_HID), lambda i: (0, i, 0)),
          pl.BlockSpec((NC, _BN, 1), lambda i: (0, i, 0)),
          pl.BlockSpec((_BN, D_HID), lambda i: (i, 0)),
          pl.BlockSpec((1, D_HID), lambda i: (0, 0)),
          pl.BlockSpec((D_HID, 2 * D_HID), lambda i: (0, 0)),
      ],
      out_specs=[
          pl.BlockSpec((_BN, D_HID), lambda i: (i, 0)),
          pl.BlockSpec((_BN, D_HID), lambda i: (i, 0)),
          pl.BlockSpec((_BN, 1), lambda i: (i, 0)),
      ],
      out_shape=[
          jax.ShapeDtypeStruct((N, D_HID), f32),
          jax.ShapeDtypeStruct((N, D_HID), f32),
          jax.ShapeDtypeStruct((N, 1), f32),
      ],
  )(agg1[:, :N], cnt1[:, :N, None], r1, b1r, w2)

  agg2 = _sc_agg(src2d, dst2d, p2, z2, z1, ones)
  if isinstance(agg2, (list, tuple)):
    agg2 = agg2[0]

  b2r = jnp.zeros((1, D_HID), f32).at[0, :D_OUT].set(b2)
  out = pl.pallas_call(
      _tc_out_kernel,
      grid=(grid,),
      in_specs=[
          pl.BlockSpec((NC, _BN, D_HID), lambda i: (0, i, 0)),
          pl.BlockSpec((_BN, 1), lambda i: (i, 0)),
          pl.BlockSpec((_BN, D_HID), lambda i: (i, 0)),
          pl.BlockSpec((1, D_HID), lambda i: (0, 0)),
      ],
      out_specs=pl.BlockSpec((_BN, D_OUT), lambda i: (i, 0)),
      out_shape=jax.ShapeDtypeStruct((N, D_OUT), f32),
  )(agg2[:, :N], iv, r2, b2r)

  return out


# speedup vs baseline: 23.6497x; 1.1856x over previous
import functools

import jax
import jax.numpy as jnp
from jax import lax
from jax.experimental import pallas as pl
from jax.experimental.pallas import tpu as pltpu
from jax.experimental.pallas import tpu_sc as plsc

N = 10000
D_IN = 128
D_HID = 16
D_OUT = 7

NC = 2
NS = 16
NW = NC * NS

NP = 10240
EB = 128
E_ROWS = 2500
RWB = E_ROWS // NW
RW_MAX = RWB + 1
ROWS_PER_TILE = NP // NS


def _sc_aggregate(with_counts):
  out_type = [jax.ShapeDtypeStruct((NC, NP, D_HID), jnp.float32)]
  if with_counts:
    out_type.append(jax.ShapeDtypeStruct((NC, NP), jnp.float32))

  scratch = [
      pltpu.VMEM((RW_MAX, EB), jnp.int32),
      pltpu.VMEM((RW_MAX, EB), jnp.int32),
      pltpu.VMEM((2, EB, D_HID), jnp.float32),
      pltpu.VMEM((EB,), jnp.float32),
      pltpu.VMEM_SHARED((NP, D_HID), jnp.float32),
      pltpu.VMEM_SHARED((NP,), jnp.float32),
      pltpu.VMEM_SHARED((N, D_HID), jnp.float32),
      pltpu.SemaphoreType.DMA,
      pltpu.SemaphoreType.DMA,
      pltpu.SemaphoreType.DMA,
  ]

  mesh = plsc.VectorSubcoreMesh(
      core_axis_name="c", subcore_axis_name="s",
      num_cores=NC, num_subcores=NS)

  @functools.partial(pl.kernel, out_type=out_type, mesh=mesh,
                     scratch_types=scratch,
                     compiler_params=pltpu.CompilerParams(
                         use_tc_tiling_on_sc=False))
  def body(ei_hbm, tbl_hbm, z2_hbm, z1_hbm, ones_hbm,
           agg_hbm, *rest):
    if with_counts:
      cnt_hbm = rest[0]
      rest = rest[1:]
    idx_s, idx_d, rows, ones_v, acc, cnta, tbls, semg, sems, semc = rest

    cid = lax.axis_index("c")
    sid = lax.axis_index("s")
    wid = cid * NS + sid

    r0 = sid * ROWS_PER_TILE
    t0 = sid * (N // NS)
    pltpu.sync_copy(tbl_hbm.at[pl.ds(t0, N // NS)],
                    tbls.at[pl.ds(t0, N // NS)])
    pltpu.sync_copy(z2_hbm.at[pl.ds(r0, ROWS_PER_TILE)],
                    acc.at[pl.ds(r0, ROWS_PER_TILE)])
    if with_counts:
      pltpu.sync_copy(z1_hbm.at[pl.ds(r0, ROWS_PER_TILE)],
                      cnta.at[pl.ds(r0, ROWS_PER_TILE)])
      pltpu.sync_copy(ones_hbm, ones_v)
    plsc.subcore_barrier()

    extra = wid < (E_ROWS - NW * RWB)
    nrows = RWB + extra.astype(jnp.int32)
    pltpu.sync_copy(ei_hbm.at[0, pl.ds(wid * RWB, RWB)],
                    idx_s.at[pl.ds(0, RWB)])
    pltpu.sync_copy(ei_hbm.at[1, pl.ds(wid * RWB, RWB)],
                    idx_d.at[pl.ds(0, RWB)])

    @pl.when(extra)
    def _():
      pltpu.sync_copy(ei_hbm.at[0, NW * RWB + wid], idx_s.at[RWB])
      pltpu.sync_copy(ei_hbm.at[1, NW * RWB + wid], idx_d.at[RWB])

    def fire_gather(j, slot):
      pltpu.async_copy(tbls.at[idx_s.at[j]], rows.at[slot], semg)

    def wait_gather():
      pltpu.make_async_copy(tbls.at[idx_s.at[0]], rows.at[0], semg).wait()

    def fire_scatter(j, slot):
      pltpu.async_copy(rows.at[slot], acc.at[idx_d.at[j]], sems, add=True)
      if with_counts:
        pltpu.async_copy(ones_v, cnta.at[idx_d.at[j]], semc, add=True)

    def wait_scatter():
      pltpu.make_async_copy(rows.at[0], acc.at[idx_d.at[0]], sems).wait()
      if with_counts:
        pltpu.make_async_copy(ones_v, cnta.at[idx_d.at[0]], semc).wait()

    fire_gather(0, 0)

    def step(j, _):
      slot = lax.rem(j, 2)
      wait_gather()

      @pl.when(j >= 1)
      def _():
        wait_scatter()

      @pl.when(j + 1 < nrows)
      def _():
        fire_gather(j + 1, 1 - slot)

      fire_scatter(j, slot)
      return 0

    lax.fori_loop(0, nrows, step, 0)
    wait_scatter()
    plsc.subcore_barrier()

    pltpu.sync_copy(acc.at[pl.ds(r0, ROWS_PER_TILE)],
                    agg_hbm.at[cid, pl.ds(r0, ROWS_PER_TILE)])
    if with_counts:
      pltpu.sync_copy(cnta.at[pl.ds(r0, ROWS_PER_TILE)],
                      cnt_hbm.at[cid, pl.ds(r0, ROWS_PER_TILE)])

  return body


_sc_agg_counts = _sc_aggregate(True)
_sc_agg = _sc_aggregate(False)


def _tc_proj_kernel(x_ref, w_ref, p_ref, r_ref):
  y = jnp.dot(x_ref[...], w_ref[...], preferred_element_type=jnp.float32)
  p_ref[...] = y[:, :D_HID]
  r_ref[...] = y[:, D_HID:]


def _tc_mid_kernel(agg_ref, cnt_ref, r1_ref, b1_ref, w2_ref,
                   p2_ref, r2_ref, iv_ref):
  agg = agg_ref[0] + agg_ref[1]
  cnt = cnt_ref[0] + cnt_ref[1]
  iv = 1.0 / jnp.maximum(cnt, 1.0)
  h = jnp.maximum(agg * iv + b1_ref[...] + r1_ref[...], 0.0)
  y = jnp.dot(h, w2_ref[...], preferred_element_type=jnp.float32)
  p2_ref[...] = y[:, :D_HID]
  r2_ref[...] = y[:, D_HID:]
  iv_ref[...] = iv


def _tc_out_kernel(agg_ref, iv_ref, r2_ref, b2_ref, o_ref):
  z = (agg_ref[0] + agg_ref[1]) * iv_ref[...] + b2_ref[...] + r2_ref[...]
  mask = lax.broadcasted_iota(jnp.int32, z.shape, 1) < D_OUT
  zm = jnp.where(mask, z, -jnp.inf)
  m = jnp.max(zm, axis=1, keepdims=True)
  e = jnp.where(mask, jnp.exp(z - m), 0.0)
  lse = jnp.log(jnp.sum(e, axis=1, keepdims=True))
  o_ref[...] = (z - m - lse)[:, :D_OUT]


_BN = 1000


def kernel(x, edge_index, W_l1, b1, W_r1, W_l2, b2, W_r2):
  f32 = jnp.float32
  ei3 = edge_index.reshape(2, E_ROWS, EB)

  z2 = jnp.zeros((NP, D_HID), f32)
  z1 = jnp.zeros((NP,), f32)
  ones = jnp.ones((EB,), f32)

  w1 = jnp.concatenate([W_l1, W_r1], axis=1)
  grid = N // _BN
  p1, r1 = pl.pallas_call(
      _tc_proj_kernel,
      grid=(grid,),
      in_specs=[
          pl.BlockSpec((_BN, D_IN), lambda i: (i, 0)),
          pl.BlockSpec((D_IN, 2 * D_HID), lambda i: (0, 0)),
      ],
      out_specs=[
          pl.BlockSpec((_BN, D_HID), lambda i: (i, 0)),
          pl.BlockSpec((_BN, D_HID), lambda i: (i, 0)),
      ],
      out_shape=[
          jax.ShapeDtypeStruct((N, D_HID), f32),
          jax.ShapeDtypeStruct((N, D_HID), f32),
      ],
  )(x, w1)

  agg1, cnt1 = _sc_agg_counts(ei3, p1, z2, z1, ones)

  wl2p = jnp.zeros((D_HID, D_HID), f32).at[:, :D_OUT].set(W_l2)
  wr2p = jnp.zeros((D_HID, D_HID), f32).at[:, :D_OUT].set(W_r2)
  w2 = jnp.concatenate([wl2p, wr2p], axis=1)
  b1r = b1.reshape(1, D_HID)
  p2, r2, iv = pl.pallas_call(
      _tc_mid_kernel,
      grid=(grid,),
      in_specs=[
          pl.BlockSpec((NC, _BN, D_HID), lambda i: (0, i, 0)),
          pl.BlockSpec((NC, _BN, 1), lambda i: (0, i, 0)),
          pl.BlockSpec((_BN, D_HID), lambda i: (i, 0)),
          pl.BlockSpec((1, D_HID), lambda i: (0, 0)),
          pl.BlockSpec((D_HID, 2 * D_HID), lambda i: (0, 0)),
      ],
      out_specs=[
          pl.BlockSpec((_BN, D_HID), lambda i: (i, 0)),
          pl.BlockSpec((_BN, D_HID), lambda i: (i, 0)),
          pl.BlockSpec((_BN, 1), lambda i: (i, 0)),
      ],
      out_shape=[
          jax.ShapeDtypeStruct((N, D_HID), f32),
          jax.ShapeDtypeStruct((N, D_HID), f32),
          jax.ShapeDtypeStruct((N, 1), f32),
      ],
  )(agg1, cnt1.reshape(NC, NP, 1), r1, b1r, w2)

  agg2 = _sc_agg(ei3, p2, z2, z1, ones)
  if isinstance(agg2, (list, tuple)):
    agg2 = agg2[0]

  b2r = jnp.zeros((1, D_HID), f32).at[0, :D_OUT].set(b2)
  out = pl.pallas_call(
      _tc_out_kernel,
      grid=(grid,),
      in_specs=[
          pl.BlockSpec((NC, _BN, D_HID), lambda i: (0, i, 0)),
          pl.BlockSpec((_BN, 1), lambda i: (i, 0)),
          pl.BlockSpec((_BN, D_HID), lambda i: (i, 0)),
          pl.BlockSpec((1, D_HID), lambda i: (0, 0)),
      ],
      out_specs=pl.BlockSpec((_BN, D_OUT), lambda i: (i, 0)),
      out_shape=jax.ShapeDtypeStruct((N, D_OUT), f32),
  )(agg2, iv, r2, b2r)

  return out
